# baseline (speedup 1.0000x reference)
import functools

import jax
import jax.numpy as jnp
from jax import lax
from jax.experimental import pallas as pl
from jax.experimental.pallas import tpu as pltpu
from jax.experimental.pallas import tpu_sc as plsc

STEPS_PER_DAY = 288
TE_DIM = 32
NUM_NODES = 170
SE_DIM = 16
OUT_DIM = SE_DIM + TE_DIM
ND = NUM_NODES * OUT_DIM
NW = 32
BT = 3072
RPW = BT // NW


def _sc_body(tod_hbm, dow_hbm, tw_hbm, dw_hbm, out_hbm,
             idx_t, idx_d, rows_t, rows_d, sem):
    wid = lax.axis_index("s") * 2 + lax.axis_index("c")
    base = wid * RPW
    pltpu.sync_copy(tod_hbm.at[pl.ds(base, RPW)], idx_t)
    pltpu.sync_copy(dow_hbm.at[pl.ds(base, RPW)], idx_d)
    for k in range(RPW // 16):
        sl = pl.ds(k * 16, 16)
        idx_t[sl] = jnp.clip(idx_t[sl], 0, STEPS_PER_DAY - 1)
        idx_d[sl] = jnp.clip(idx_d[sl], 0, 6)
    pltpu.async_copy(tw_hbm.at[idx_t], rows_t, sem).wait()
    pltpu.async_copy(dw_hbm.at[idx_d], rows_d, sem).wait()
    for r in range(RPW):
        for c in range(TE_DIM // 16):
            sl = pl.ds(c * 16, 16)
            rows_t[r, sl] = rows_t[r, sl] + rows_d[r, sl]
    pltpu.sync_copy(rows_t, out_hbm.at[pl.ds(base, RPW)])


def _tc_body(te_ref, se_ref, a2_ref, o_ref):
    te_t = te_ref[0, :, :TE_DIM]
    o_ref[0] = lax.dot_general(
        a2_ref[...], te_t,
        dimension_numbers=(((1,), (1,)), ((), ())),
        preferred_element_type=jnp.float32) + se_ref[...]


@jax.jit
def kernel(te, se, tod_weight, dow_weight):
    b, t = te.shape[0], te.shape[1]
    tod_ids = te[..., 0].astype(jnp.int32).T.reshape(BT)
    dow_ids = te[..., 1].astype(jnp.int32).T.reshape(BT)
    tod_w128 = jnp.pad(tod_weight, ((0, 0), (0, 128 - TE_DIM)))
    dow_w128 = jnp.pad(dow_weight, ((0, 1), (0, 128 - TE_DIM)))
    se_col = jnp.pad(se, ((0, 0), (0, TE_DIM))).reshape(ND, 1)
    a2mat = jnp.tile(
        jnp.concatenate([jnp.zeros((SE_DIM, TE_DIM), jnp.float32),
                         jnp.eye(TE_DIM, dtype=jnp.float32)], axis=0),
        (NUM_NODES, 1))

    mesh = plsc.VectorSubcoreMesh(core_axis_name="c", subcore_axis_name="s")
    sc_gather = functools.partial(
        pl.kernel, mesh=mesh,
        out_type=jax.ShapeDtypeStruct((BT, 128), jnp.float32),
        scratch_types=[
            pltpu.VMEM((RPW,), jnp.int32),
            pltpu.VMEM((RPW,), jnp.int32),
            pltpu.VMEM((RPW, 128), jnp.float32),
            pltpu.VMEM((RPW, 128), jnp.float32),
            pltpu.SemaphoreType.DMA,
        ],
    )(_sc_body)
    te_emb = sc_gather(tod_ids, dow_ids, tod_w128, dow_w128)
    te_emb = te_emb.reshape(t, b, 128)

    out = pl.pallas_call(
        _tc_body,
        grid=(t,),
        in_specs=[
            pl.BlockSpec((1, b, 128), lambda i: (i, 0, 0)),
            pl.BlockSpec((ND, 1), lambda i: (0, 0)),
            pl.BlockSpec((ND, TE_DIM), lambda i: (0, 0)),
        ],
        out_specs=pl.BlockSpec((1, ND, b), lambda i: (i, 0, 0)),
        out_shape=jax.ShapeDtypeStruct((t, ND, b), jnp.float32),
    )(te_emb, se_col, a2mat)
    out = out.reshape(t, NUM_NODES, OUT_DIM, b)
    return jnp.transpose(out, (3, 0, 1, 2))


# speedup vs baseline: 3.6808x; 3.6808x over previous
import jax
import jax.numpy as jnp
from jax import lax
from jax.experimental import pallas as pl

STEPS_PER_DAY = 288
TE_DIM = 32
NUM_NODES = 170
SE_DIM = 16
OUT_DIM = SE_DIM + TE_DIM
ND = NUM_NODES * OUT_DIM


NSPLIT = 2
NDH = ND // NSPLIT


def _body(tod_ref, dow_ref, se_ref, a_ref, tw_ref, dw_ref, o_ref):
    bsz = tod_ref.shape[-1]
    j = pl.program_id(1)
    tod = jnp.clip(tod_ref[0], 0, STEPS_PER_DAY - 1)
    dow = jnp.clip(dow_ref[0], 0, 6)
    oh_t = (tod == lax.broadcasted_iota(jnp.int32, (STEPS_PER_DAY, bsz), 0))
    oh_d = (dow == lax.broadcasted_iota(jnp.int32, (8, bsz), 0))
    te_t = lax.dot(tw_ref[...], oh_t.astype(jnp.float32),
                   preferred_element_type=jnp.float32)
    te_t = te_t + lax.dot(dw_ref[...], oh_d.astype(jnp.float32),
                          preferred_element_type=jnp.float32)
    bmat = jnp.concatenate(
        [jnp.zeros((SE_DIM, bsz), jnp.float32), te_t], axis=0)
    row0 = j * NDH
    o_ref[0] = lax.dot(a_ref[pl.ds(row0, NDH), :], bmat,
                       preferred_element_type=jnp.float32
                       ) + se_ref[pl.ds(row0, NDH), :]


@jax.jit
def kernel(te, se, tod_weight, dow_weight):
    b, t = te.shape[0], te.shape[1]
    tod_ids = te[..., 0].astype(jnp.int32).T.reshape(t, 1, b)
    dow_ids = te[..., 1].astype(jnp.int32).T.reshape(t, 1, b)
    tod_wT = tod_weight.T
    dow_wT = jnp.pad(dow_weight, ((0, 1), (0, 0))).T
    se_col = jnp.pad(se, ((0, 0), (0, TE_DIM))).reshape(ND, 1)
    amat = jnp.tile(jnp.eye(OUT_DIM, dtype=jnp.float32),
                    (NUM_NODES, 1))

    out = pl.pallas_call(
        _body,
        grid=(t, NSPLIT),
        in_specs=[
            pl.BlockSpec((1, 1, b), lambda i, j: (i, 0, 0)),
            pl.BlockSpec((1, 1, b), lambda i, j: (i, 0, 0)),
            pl.BlockSpec((ND, 1), lambda i, j: (0, 0)),
            pl.BlockSpec((ND, OUT_DIM), lambda i, j: (0, 0)),
            pl.BlockSpec((TE_DIM, STEPS_PER_DAY), lambda i, j: (0, 0)),
            pl.BlockSpec((TE_DIM, 8), lambda i, j: (0, 0)),
        ],
        out_specs=pl.BlockSpec((1, NDH, b), lambda i, j: (i, j, 0)),
        out_shape=jax.ShapeDtypeStruct((t, ND, b), jnp.float32),
    )(tod_ids, dow_ids, se_col, amat, tod_wT, dow_wT)
    out = out.reshape(t, NUM_NODES, OUT_DIM, b)
    return jnp.transpose(out, (3, 0, 1, 2))


# speedup vs baseline: 3.8219x; 1.0383x over previous
import jax
import jax.numpy as jnp
from jax import lax
from jax.experimental import pallas as pl

STEPS_PER_DAY = 288
TE_DIM = 32
NUM_NODES = 170
SE_DIM = 16
OUT_DIM = SE_DIM + TE_DIM
ND = NUM_NODES * OUT_DIM
TPT = 2


def _body(tod_ref, dow_ref, se_ref, a_ref, tw_ref, dw_ref, o_ref):
    bsz = tod_ref.shape[-1]
    for k in range(TPT):
        tod = jnp.clip(tod_ref[0, k:k + 1], 0, STEPS_PER_DAY - 1)
        dow = jnp.clip(dow_ref[0, k:k + 1], 0, 6)
        oh_t = (tod == lax.broadcasted_iota(jnp.int32,
                                            (STEPS_PER_DAY, bsz), 0))
        oh_d = (dow == lax.broadcasted_iota(jnp.int32, (8, bsz), 0))
        te_t = lax.dot(tw_ref[...], oh_t.astype(jnp.float32),
                       preferred_element_type=jnp.float32)
        te_t = te_t + lax.dot(dw_ref[...], oh_d.astype(jnp.float32),
                              preferred_element_type=jnp.float32)
        bmat = jnp.concatenate(
            [jnp.zeros((SE_DIM, bsz), jnp.float32), te_t], axis=0)
        o_ref[k] = lax.dot(a_ref[...], bmat,
                           preferred_element_type=jnp.float32) + se_ref[...]


@jax.jit
def kernel(te, se, tod_weight, dow_weight):
    b, t = te.shape[0], te.shape[1]
    tod_ids = te[..., 0].astype(jnp.int32).T.reshape(t // TPT, TPT, b)
    dow_ids = te[..., 1].astype(jnp.int32).T.reshape(t // TPT, TPT, b)
    tod_wT = tod_weight.T
    dow_wT = jnp.pad(dow_weight, ((0, 1), (0, 0))).T
    se_col = jnp.pad(se, ((0, 0), (0, TE_DIM))).reshape(ND, 1)
    amat = jnp.tile(jnp.eye(OUT_DIM, dtype=jnp.float32),
                    (NUM_NODES, 1))

    out = pl.pallas_call(
        _body,
        grid=(t // TPT,),
        in_specs=[
            pl.BlockSpec((1, TPT, b), lambda i: (i, 0, 0)),
            pl.BlockSpec((1, TPT, b), lambda i: (i, 0, 0)),
            pl.BlockSpec((ND, 1), lambda i: (0, 0)),
            pl.BlockSpec((ND, OUT_DIM), lambda i: (0, 0)),
            pl.BlockSpec((TE_DIM, STEPS_PER_DAY), lambda i: (0, 0)),
            pl.BlockSpec((TE_DIM, 8), lambda i: (0, 0)),
        ],
        out_specs=pl.BlockSpec((TPT, ND, b), lambda i: (i, 0, 0)),
        out_shape=jax.ShapeDtypeStruct((t, ND, b), jnp.float32),
    )(tod_ids, dow_ids, se_col, amat, tod_wT, dow_wT)
    out = out.reshape(t, NUM_NODES, OUT_DIM, b)
    return jnp.transpose(out, (3, 0, 1, 2))


# speedup vs baseline: 3.9596x; 1.0360x over previous
import jax
import jax.numpy as jnp
from jax import lax
from jax.experimental import pallas as pl

STEPS_PER_DAY = 288
TE_DIM = 32
NUM_NODES = 170
SE_DIM = 16
OUT_DIM = SE_DIM + TE_DIM
ND = NUM_NODES * OUT_DIM


def _body(tod_ref, dow_ref, se_ref, a_ref, tw_ref, dw_ref, o_ref):
    bsz = tod_ref.shape[-1]
    tod = jnp.clip(tod_ref[0], 0, STEPS_PER_DAY - 1)
    dow = jnp.clip(dow_ref[0], 0, 6)
    oh_t = (tod == lax.broadcasted_iota(jnp.int32, (STEPS_PER_DAY, bsz), 0))
    oh_d = (dow == lax.broadcasted_iota(jnp.int32, (8, bsz), 0))
    te_t = lax.dot(tw_ref[...], oh_t.astype(jnp.float32),
                   preferred_element_type=jnp.float32)
    te_t = te_t + lax.dot(dw_ref[...], oh_d.astype(jnp.float32),
                          preferred_element_type=jnp.float32)
    bmat = jnp.concatenate(
        [jnp.zeros((SE_DIM, bsz), jnp.float32), te_t], axis=0)
    o_ref[0] = lax.dot(a_ref[...], bmat,
                       preferred_element_type=jnp.float32) + se_ref[...]


@jax.jit
def kernel(te, se, tod_weight, dow_weight):
    b, t = te.shape[0], te.shape[1]
    tod_ids = te[..., 0].astype(jnp.int32).T.reshape(t, 1, b)
    dow_ids = te[..., 1].astype(jnp.int32).T.reshape(t, 1, b)
    tod_wT = tod_weight.T
    dow_wT = jnp.pad(dow_weight, ((0, 1), (0, 0))).T
    se_col = jnp.pad(se, ((0, 0), (0, TE_DIM))).reshape(ND, 1)
    amat = jnp.tile(jnp.eye(OUT_DIM, dtype=jnp.float32),
                    (NUM_NODES, 1))

    out = pl.pallas_call(
        _body,
        grid=(t,),
        in_specs=[
            pl.BlockSpec((1, 1, b), lambda i: (i, 0, 0)),
            pl.BlockSpec((1, 1, b), lambda i: (i, 0, 0)),
            pl.BlockSpec((ND, 1), lambda i: (0, 0)),
            pl.BlockSpec((ND, OUT_DIM), lambda i: (0, 0)),
            pl.BlockSpec((TE_DIM, STEPS_PER_DAY), lambda i: (0, 0)),
            pl.BlockSpec((TE_DIM, 8), lambda i: (0, 0)),
        ],
        out_specs=pl.BlockSpec((1, ND, b), lambda i: (i, 0, 0)),
        out_shape=jax.ShapeDtypeStruct((t, ND, b), jnp.float32),
    )(tod_ids, dow_ids, se_col, amat, tod_wT, dow_wT)
    out = out.reshape(t, NUM_NODES, OUT_DIM, b)
    return jnp.transpose(out, (3, 0, 1, 2))


# speedup vs baseline: 4.1512x; 1.0484x over previous
import jax
import jax.numpy as jnp
from jax import lax
from jax.experimental import pallas as pl
from jax.experimental.pallas import tpu as pltpu

STEPS_PER_DAY = 288
TE_DIM = 32
NUM_NODES = 170
SE_DIM = 16
OUT_DIM = SE_DIM + TE_DIM
ND = NUM_NODES * OUT_DIM


def _body(tod_ref, dow_ref, se_ref, tw_ref, dw_ref, o_ref, a_ref):
    bsz = tod_ref.shape[-1]

    @pl.when(pl.program_id(0) == 0)
    def _init_a():
        r = lax.broadcasted_iota(jnp.int32, (ND, OUT_DIM), 0)
        c = lax.broadcasted_iota(jnp.int32, (ND, OUT_DIM), 1)
        a_ref[...] = (lax.rem(r, OUT_DIM) == c).astype(jnp.float32)

    tod = jnp.clip(tod_ref[0], 0, STEPS_PER_DAY - 1)
    dow = jnp.clip(dow_ref[0], 0, 6)
    oh_t = (tod == lax.broadcasted_iota(jnp.int32, (STEPS_PER_DAY, bsz), 0))
    oh_d = (dow == lax.broadcasted_iota(jnp.int32, (8, bsz), 0))
    te_t = lax.dot(tw_ref[...], oh_t.astype(jnp.float32),
                   preferred_element_type=jnp.float32)
    te_t = te_t + lax.dot(dw_ref[...], oh_d.astype(jnp.float32),
                          preferred_element_type=jnp.float32)
    bmat = jnp.concatenate(
        [jnp.zeros((SE_DIM, bsz), jnp.float32), te_t], axis=0)
    o_ref[0] = lax.dot(a_ref[...], bmat,
                       preferred_element_type=jnp.float32) + se_ref[...]


@jax.jit
def kernel(te, se, tod_weight, dow_weight):
    b, t = te.shape[0], te.shape[1]
    tod_ids = te[..., 0].astype(jnp.int32).T.reshape(t, 1, b)
    dow_ids = te[..., 1].astype(jnp.int32).T.reshape(t, 1, b)
    tod_wT = tod_weight.T
    dow_wT = jnp.pad(dow_weight, ((0, 1), (0, 0))).T
    se_col = jnp.pad(se, ((0, 0), (0, TE_DIM))).reshape(ND, 1)

    out = pl.pallas_call(
        _body,
        grid=(t,),
        in_specs=[
            pl.BlockSpec((1, 1, b), lambda i: (i, 0, 0)),
            pl.BlockSpec((1, 1, b), lambda i: (i, 0, 0)),
            pl.BlockSpec((ND, 1), lambda i: (0, 0)),
            pl.BlockSpec((TE_DIM, STEPS_PER_DAY), lambda i: (0, 0)),
            pl.BlockSpec((TE_DIM, 8), lambda i: (0, 0)),
        ],
        out_specs=pl.BlockSpec((1, ND, b), lambda i: (i, 0, 0)),
        out_shape=jax.ShapeDtypeStruct((t, ND, b), jnp.float32),
        scratch_shapes=[pltpu.VMEM((ND, OUT_DIM), jnp.float32)],
    )(tod_ids, dow_ids, se_col, tod_wT, dow_wT)
    out = out.reshape(t, NUM_NODES, OUT_DIM, b)
    return jnp.transpose(out, (3, 0, 1, 2))
